# baseline (speedup 1.0000x reference)
import jax
import jax.numpy as jnp
from jax.experimental import pallas as pl

K = 32


def _enc_topk_body(x_ref, we_ref, be_ref, sp_ref, *, ht: int):
    j = pl.program_id(1)
    nj = pl.num_programs(1)
    h = jax.lax.dot_general(
        x_ref[...], we_ref[...], (((1,), (1,)), ((), ())),
        preferred_element_type=jnp.float32)
    h = jnp.maximum(h + be_ref[...].astype(jnp.float32), 0.0)
    sp_ref[:, pl.ds(j * ht, ht)] = h

    @pl.when(j == nj - 1)
    def _():
        vk = jnp.max(sp_ref[...], axis=1, keepdims=True)
        for _ in range(K - 1):
            hh = sp_ref[...]
            vk = jnp.max(jnp.where(hh < vk, hh, -1.0), axis=1,
                         keepdims=True)
        thr = jnp.maximum(vk, 0.0)
        hh = sp_ref[...]
        sp_ref[...] = jnp.where(hh >= thr, hh, 0.0)


def _decode_body(sp_ref, wd_ref, bd_ref, out_ref):
    j = pl.program_id(1)
    part = jax.lax.dot_general(
        sp_ref[...].astype(jnp.bfloat16), wd_ref[...], (((1,), (1,)), ((), ())),
        preferred_element_type=jnp.float32)

    @pl.when(j == 0)
    def _():
        out_ref[...] = part + bd_ref[...]

    @pl.when(j != 0)
    def _():
        out_ref[...] = out_ref[...] + part


def kernel(x, W_enc, b_enc, W_dec, b_dec):
    B, D = x.shape
    H = W_enc.shape[0]

    bt = min(128, B)
    ht = min(2048, H)
    assert B % bt == 0 and H % ht == 0

    import functools
    sparse_hidden = pl.pallas_call(
        functools.partial(_enc_topk_body, ht=ht),
        grid=(B // bt, H // ht),
        in_specs=[
            pl.BlockSpec((bt, D), lambda i, j: (i, 0)),
            pl.BlockSpec((ht, D), lambda i, j: (j, 0)),
            pl.BlockSpec((1, ht), lambda i, j: (0, j)),
        ],
        out_specs=pl.BlockSpec((bt, H), lambda i, j: (i, 0)),
        out_shape=jax.ShapeDtypeStruct((B, H), jnp.float32),
    )(x.astype(jnp.bfloat16), W_enc.astype(jnp.bfloat16), b_enc.reshape(1, H))

    bt2 = min(512, B)
    ht2 = min(2048, H)
    reconstructed = pl.pallas_call(
        _decode_body,
        grid=(B // bt2, H // ht2),
        in_specs=[
            pl.BlockSpec((bt2, ht2), lambda i, j: (i, j)),
            pl.BlockSpec((D, ht2), lambda i, j: (0, j)),
            pl.BlockSpec((1, D), lambda i, j: (0, 0)),
        ],
        out_specs=pl.BlockSpec((bt2, D), lambda i, j: (i, 0)),
        out_shape=jax.ShapeDtypeStruct((B, D), jnp.float32),
    )(sparse_hidden, W_dec.astype(jnp.bfloat16), b_dec.reshape(1, D))

    return (reconstructed, sparse_hidden)


# speedup vs baseline: 1.3491x; 1.3491x over previous
import jax
import jax.numpy as jnp
from jax.experimental import pallas as pl

K = 32


def _enc_topk_body(x_ref, we_ref, be_ref, sp_ref, *, ht: int):
    j = pl.program_id(1)
    nj = pl.num_programs(1)
    h = jax.lax.dot_general(
        x_ref[...], we_ref[...], (((1,), (1,)), ((), ())),
        preferred_element_type=jnp.float32)
    h = jnp.maximum(h + be_ref[...].astype(jnp.float32), 0.0)
    sp_ref[:, pl.ds(j * ht, ht)] = h

    @pl.when(j == nj - 1)
    def _():
        m0 = jnp.max(sp_ref[...], axis=1, keepdims=True)
        lo0 = jnp.zeros(m0.shape, jnp.uint32)
        hi0 = jax.lax.bitcast_convert_type(m0, jnp.uint32) + jnp.uint32(1)

        def cond(s):
            lo, hi = s
            return jnp.any(hi - lo > jnp.uint32(1))

        def body(s):
            lo, hi = s
            mid = lo + ((hi - lo) >> jnp.uint32(1))
            hb = jax.lax.bitcast_convert_type(sp_ref[...], jnp.uint32)
            c = jnp.sum(jnp.where(hb >= mid, 1.0, 0.0), axis=1,
                        keepdims=True)
            ge = c >= float(K)
            return (jnp.where(ge, mid, lo), jnp.where(ge, hi, mid))

        lo, _ = jax.lax.while_loop(cond, body, (lo0, hi0))
        thr = jax.lax.bitcast_convert_type(lo, jnp.float32)
        hh = sp_ref[...]
        sp_ref[...] = jnp.where(hh >= thr, hh, 0.0)


def _decode_body(sp_ref, wd_ref, bd_ref, out_ref):
    j = pl.program_id(1)
    part = jax.lax.dot_general(
        sp_ref[...].astype(jnp.bfloat16), wd_ref[...], (((1,), (1,)), ((), ())),
        preferred_element_type=jnp.float32)

    @pl.when(j == 0)
    def _():
        out_ref[...] = part + bd_ref[...]

    @pl.when(j != 0)
    def _():
        out_ref[...] = out_ref[...] + part


def kernel(x, W_enc, b_enc, W_dec, b_dec):
    B, D = x.shape
    H = W_enc.shape[0]

    bt = min(128, B)
    ht = min(2048, H)
    assert B % bt == 0 and H % ht == 0

    import functools
    sparse_hidden = pl.pallas_call(
        functools.partial(_enc_topk_body, ht=ht),
        grid=(B // bt, H // ht),
        in_specs=[
            pl.BlockSpec((bt, D), lambda i, j: (i, 0)),
            pl.BlockSpec((ht, D), lambda i, j: (j, 0)),
            pl.BlockSpec((1, ht), lambda i, j: (0, j)),
        ],
        out_specs=pl.BlockSpec((bt, H), lambda i, j: (i, 0)),
        out_shape=jax.ShapeDtypeStruct((B, H), jnp.float32),
    )(x.astype(jnp.bfloat16), W_enc.astype(jnp.bfloat16), b_enc.reshape(1, H))

    bt2 = min(512, B)
    ht2 = min(2048, H)
    reconstructed = pl.pallas_call(
        _decode_body,
        grid=(B // bt2, H // ht2),
        in_specs=[
            pl.BlockSpec((bt2, ht2), lambda i, j: (i, j)),
            pl.BlockSpec((D, ht2), lambda i, j: (0, j)),
            pl.BlockSpec((1, D), lambda i, j: (0, 0)),
        ],
        out_specs=pl.BlockSpec((bt2, D), lambda i, j: (i, 0)),
        out_shape=jax.ShapeDtypeStruct((B, D), jnp.float32),
    )(sparse_hidden, W_dec.astype(jnp.bfloat16), b_dec.reshape(1, D))

    return (reconstructed, sparse_hidden)


# speedup vs baseline: 1.6092x; 1.1928x over previous
import functools

import jax
import jax.numpy as jnp
from jax.experimental import pallas as pl
from jax.experimental.pallas import tpu as pltpu

K = 32


def _enc_topk_body(x_ref, we_ref, be_ref, sp_ref, g_ref, *, ht: int):
    j = pl.program_id(1)
    nj = pl.num_programs(1)
    h = jax.lax.dot_general(
        x_ref[...], we_ref[...], (((1,), (1,)), ((), ())),
        preferred_element_type=jnp.float32)
    h = jnp.maximum(h + be_ref[...].astype(jnp.float32), 0.0)
    sp_ref[:, pl.ds(j * ht, ht)] = h

    g = h[:, 0:128]
    for k in range(1, ht // 128):
        g = jnp.maximum(g, h[:, k * 128:(k + 1) * 128])

    @pl.when(j == 0)
    def _():
        g_ref[...] = g

    @pl.when(j > 0)
    def _():
        g_ref[...] = jnp.maximum(g_ref[...], g)

    @pl.when(j == nj - 1)
    def _():
        gg = g_ref[...]
        m0 = jnp.max(gg, axis=1, keepdims=True)
        vk = m0
        for _ in range(K - 1):
            vk = jnp.max(jnp.where(gg < vk, gg, -1.0), axis=1, keepdims=True)
        tau = jnp.maximum(vk, 0.0)

        lo0 = jax.lax.bitcast_convert_type(tau, jnp.uint32)
        hi0 = jax.lax.bitcast_convert_type(m0, jnp.uint32) + jnp.uint32(1)

        def cond(s):
            lo, hi = s
            return jnp.any(hi - lo > jnp.uint32(1))

        def body(s):
            lo, hi = s
            mid = lo + ((hi - lo) >> jnp.uint32(1))
            hb = jax.lax.bitcast_convert_type(sp_ref[...], jnp.uint32)
            c = jnp.sum(jnp.where(hb >= mid, 1.0, 0.0), axis=1,
                        keepdims=True)
            exact = c == float(K)
            ge = c >= float(K)
            lo2 = jnp.where(ge, mid, lo)
            hi2 = jnp.where(exact, mid + jnp.uint32(1),
                            jnp.where(ge, hi, mid))
            return (lo2, hi2)

        lo, _ = jax.lax.while_loop(cond, body, (lo0, hi0))
        thr = jax.lax.bitcast_convert_type(lo, jnp.float32)
        hh = sp_ref[...]
        sp_ref[...] = jnp.where(hh >= thr, hh, 0.0)


def _decode_body(sp_ref, wd_ref, bd_ref, out_ref):
    j = pl.program_id(1)
    part = jax.lax.dot_general(
        sp_ref[...].astype(jnp.bfloat16), wd_ref[...], (((1,), (1,)), ((), ())),
        preferred_element_type=jnp.float32)

    @pl.when(j == 0)
    def _():
        out_ref[...] = part + bd_ref[...]

    @pl.when(j != 0)
    def _():
        out_ref[...] = out_ref[...] + part


def kernel(x, W_enc, b_enc, W_dec, b_dec):
    B, D = x.shape
    H = W_enc.shape[0]

    bt = min(128, B)
    ht = min(2048, H)
    assert B % bt == 0 and H % ht == 0

    sparse_hidden = pl.pallas_call(
        functools.partial(_enc_topk_body, ht=ht),
        grid=(B // bt, H // ht),
        in_specs=[
            pl.BlockSpec((bt, D), lambda i, j: (i, 0)),
            pl.BlockSpec((ht, D), lambda i, j: (j, 0)),
            pl.BlockSpec((1, ht), lambda i, j: (0, j)),
        ],
        out_specs=pl.BlockSpec((bt, H), lambda i, j: (i, 0)),
        out_shape=jax.ShapeDtypeStruct((B, H), jnp.float32),
        scratch_shapes=[pltpu.VMEM((bt, 128), jnp.float32)],
    )(x.astype(jnp.bfloat16), W_enc.astype(jnp.bfloat16), b_enc.reshape(1, H))

    bt2 = min(512, B)
    ht2 = min(2048, H)
    reconstructed = pl.pallas_call(
        _decode_body,
        grid=(B // bt2, H // ht2),
        in_specs=[
            pl.BlockSpec((bt2, ht2), lambda i, j: (i, j)),
            pl.BlockSpec((D, ht2), lambda i, j: (0, j)),
            pl.BlockSpec((1, D), lambda i, j: (0, 0)),
        ],
        out_specs=pl.BlockSpec((bt2, D), lambda i, j: (i, 0)),
        out_shape=jax.ShapeDtypeStruct((B, D), jnp.float32),
    )(sparse_hidden, W_dec.astype(jnp.bfloat16), b_dec.reshape(1, D))

    return (reconstructed, sparse_hidden)
